# baseline (speedup 1.0000x reference)
import functools

import jax
import jax.numpy as jnp
from jax import lax
from jax.experimental import pallas as pl
from jax.experimental.pallas import tpu as pltpu
from jax.experimental.pallas import tpu_sc as plsc

BATCH = 16384
D_IN = 128
FEAT_DIM = 64

_NC = 2
_NS = 16
_NW = _NC * _NS
_BPW = BATCH // _NW


def _sc_gather_body(idx_hbm, table_hbm, emb_hbm, idx_v, rows_v, sem):
    wid = lax.axis_index("s") * _NC + lax.axis_index("c")
    base = wid * _BPW
    pltpu.sync_copy(idx_hbm.at[pl.ds(base, _BPW)], idx_v)
    pltpu.async_copy(table_hbm.at[idx_v], rows_v, sem).wait()
    pltpu.sync_copy(rows_v, emb_hbm.at[pl.ds(base, _BPW)])


@functools.partial(
    pl.kernel,
    out_type=jax.ShapeDtypeStruct((BATCH, FEAT_DIM), jnp.float32),
    mesh=plsc.VectorSubcoreMesh(core_axis_name="c", subcore_axis_name="s"),
    compiler_params=pltpu.CompilerParams(use_tc_tiling_on_sc=False),
    scratch_types=[
        pltpu.VMEM((_BPW,), jnp.int32),
        pltpu.VMEM((_BPW, FEAT_DIM), jnp.float32),
        pltpu.SemaphoreType.DMA,
    ],
)
def _sc_gather(idx_hbm, table_hbm, emb_hbm, idx_v, rows_v, sem):
    _sc_gather_body(idx_hbm, table_hbm, emb_hbm, idx_v, rows_v, sem)


def _tc_body(feat_ref, w_ref, b_ref, emb_ref, out_ref):
    proj = (
        lax.dot_general(
            feat_ref[...], w_ref[...],
            (((1,), (1,)), ((), ())),
            preferred_element_type=jnp.float32,
        )
        + b_ref[...]
    )
    out_ref[...] = jnp.concatenate([proj, emb_ref[...]], axis=-1)


def _tc_project_concat(feat, W, b, emb):
    blk = 2048
    grid = BATCH // blk
    return pl.pallas_call(
        _tc_body,
        grid=(grid,),
        in_specs=[
            pl.BlockSpec((blk, D_IN), lambda i: (i, 0)),
            pl.BlockSpec((FEAT_DIM, D_IN), lambda i: (0, 0)),
            pl.BlockSpec((1, FEAT_DIM), lambda i: (0, 0)),
            pl.BlockSpec((blk, FEAT_DIM), lambda i: (i, 0)),
        ],
        out_specs=pl.BlockSpec((blk, D_IN), lambda i: (i, 0)),
        out_shape=jax.ShapeDtypeStruct((BATCH, D_IN), jnp.float32),
    )(feat, W, b.reshape(1, FEAT_DIM), emb)


def kernel(feat, idx, W, b, table):
    emb = _sc_gather(idx.astype(jnp.int32), table)
    return _tc_project_concat(feat, W, b, emb)
